# initial kernel scaffold (unmeasured)
import jax
import jax.numpy as jnp
from jax import lax
from jax.experimental import pallas as pl
from jax.experimental.pallas import tpu as pltpu

N_DEV = 4

COMM_INT8 = True
DOT_DTYPE = jnp.float8_e5m2


def kernel(x, w_mat, scale_x, scale_w):
    m, _ = x.shape
    _, n = w_mat.shape
    mc = m // N_DEV

    qdtype = jnp.int8 if COMM_INT8 else jnp.float32

    def quantize(v):
        if not COMM_INT8:
            return v, jnp.float32(1.0)
        scale = jnp.maximum(jnp.max(jnp.abs(v)), 1e-30) / 127.0
        q = jnp.clip(jnp.round(v / scale), -127.0, 127.0).astype(jnp.int8)
        return q, scale

    def body(x_ref, w_ref, sx_ref, sw_ref, out_ref,
             acc_ref, s1s_ref, s1r_ref, s2s_ref, s2r_ref,
             scs_ref, scr_ref, send_sems, recv_sems):
        my_i = lax.axis_index("i")
        p1 = jnp.bitwise_xor(my_i, 1)
        p2 = (N_DEV - 1) - my_i

        ka = jnp.minimum(my_i, p2)
        kb = jnp.maximum(my_i, p2)
        sa = jnp.minimum(p1, (N_DEV - 1) - p1)
        sb = jnp.maximum(p1, (N_DEV - 1) - p1)

        w_q = w_ref[:, :].astype(DOT_DTYPE)

        def pchunk(c):
            a = x_ref[pl.ds(c * mc, mc), :].astype(DOT_DTYPE)
            return jnp.dot(a, w_q, preferred_element_type=jnp.float32)

        psa = pchunk(sa)
        psb = pchunk(sb)
        if COMM_INT8:
            amax = jnp.maximum(jnp.max(jnp.abs(psa)), jnp.max(jnp.abs(psb)))
            sc1 = jnp.maximum(amax, 1e-30) / 127.0
            s1s_ref[0, :, :] = jnp.clip(
                jnp.round(psa / sc1), -127.0, 127.0).astype(jnp.int8)
            s1s_ref[1, :, :] = jnp.clip(
                jnp.round(psb / sc1), -127.0, 127.0).astype(jnp.int8)
        else:
            sc1 = jnp.float32(1.0)
            s1s_ref[0, :, :] = psa
            s1s_ref[1, :, :] = psb
        scs_ref[0, :, :] = jnp.full((8, 128), sc1, jnp.float32)

        barrier = pltpu.get_barrier_semaphore()
        pl.semaphore_signal(barrier, inc=1, device_id=(p1,),
                            device_id_type=pl.DeviceIdType.MESH)
        pl.semaphore_signal(barrier, inc=1, device_id=(p2,),
                            device_id_type=pl.DeviceIdType.MESH)
        pl.semaphore_wait(barrier, 2)

        rdma1 = pltpu.make_async_remote_copy(
            src_ref=s1s_ref, dst_ref=s1r_ref,
            send_sem=send_sems.at[0], recv_sem=recv_sems.at[0],
            device_id=(p1,), device_id_type=pl.DeviceIdType.MESH)
        rdma1.start()
        rdma1_sc = pltpu.make_async_remote_copy(
            src_ref=scs_ref.at[0], dst_ref=scr_ref.at[0],
            send_sem=send_sems.at[1], recv_sem=recv_sems.at[1],
            device_id=(p1,), device_id_type=pl.DeviceIdType.MESH)
        rdma1_sc.start()

        acc_ref[0, :, :] = pchunk(ka)
        acc_ref[1, :, :] = pchunk(kb)

        rdma1.wait()
        rdma1_sc.wait()
        sc1r = scr_ref[0, 0, 0]
        acc_ref[0, :, :] = acc_ref[0] + s1r_ref[0].astype(jnp.float32) * sc1r
        acc_ref[1, :, :] = acc_ref[1] + s1r_ref[1].astype(jnp.float32) * sc1r

        slot2 = jnp.where(my_i < 2, 1, 0)
        v2 = acc_ref[pl.ds(slot2, 1), :, :][0]
        q2, sc2 = quantize(v2)
        s2s_ref[:, :] = q2
        scs_ref[1, :, :] = jnp.full((8, 128), sc2, jnp.float32)

        rdma2 = pltpu.make_async_remote_copy(
            src_ref=s2s_ref, dst_ref=s2r_ref,
            send_sem=send_sems.at[2], recv_sem=recv_sems.at[2],
            device_id=(p2,), device_id_type=pl.DeviceIdType.MESH)
        rdma2.start()
        rdma2_sc = pltpu.make_async_remote_copy(
            src_ref=scs_ref.at[1], dst_ref=scr_ref.at[1],
            send_sem=send_sems.at[3], recv_sem=recv_sems.at[3],
            device_id=(p2,), device_id_type=pl.DeviceIdType.MESH)
        rdma2_sc.start()
        rdma2.wait()
        rdma2_sc.wait()

        keep = acc_ref[pl.ds(1 - slot2, 1), :, :][0]
        sc2r = scr_ref[1, 0, 0]
        out_ref[:, :] = (keep + s2r_ref[:, :].astype(jnp.float32) * sc2r) * (
            sx_ref[0] * sw_ref[0])

    return pl.pallas_call(
        body,
        out_shape=jax.ShapeDtypeStruct((mc, n), jnp.float32),
        in_specs=[
            pl.BlockSpec(memory_space=pltpu.VMEM),
            pl.BlockSpec(memory_space=pltpu.VMEM),
            pl.BlockSpec(memory_space=pltpu.SMEM),
            pl.BlockSpec(memory_space=pltpu.SMEM),
        ],
        out_specs=pl.BlockSpec(memory_space=pltpu.VMEM),
        scratch_shapes=[
            pltpu.VMEM((2, mc, n), jnp.float32),
            pltpu.VMEM((2, mc, n), qdtype),
            pltpu.VMEM((2, mc, n), qdtype),
            pltpu.VMEM((mc, n), qdtype),
            pltpu.VMEM((mc, n), qdtype),
            pltpu.VMEM((2, 8, 128), jnp.float32),
            pltpu.VMEM((2, 8, 128), jnp.float32),
            pltpu.SemaphoreType.DMA((4,)),
            pltpu.SemaphoreType.DMA((4,)),
        ],
        compiler_params=pltpu.CompilerParams(collective_id=0),
    )(x, w_mat, scale_x, scale_w)


# baseline (device time: 107900 ns/iter reference)
import jax
import jax.numpy as jnp
from jax import lax
from jax.experimental import pallas as pl
from jax.experimental.pallas import tpu as pltpu

N_DEV = 4

COMM_INT8 = True
DOT_DTYPE = jnp.float8_e5m2


def kernel(x, w_mat, scale_x, scale_w):
    m, _ = x.shape
    _, n = w_mat.shape
    mc = m // N_DEV

    qdtype = jnp.int8 if COMM_INT8 else jnp.float32

    def quantize(v):
        if not COMM_INT8:
            return v, jnp.float32(1.0)
        scale = jnp.maximum(jnp.max(jnp.abs(v)), 1e-30) / 127.0
        q = jnp.clip(jnp.round(v / scale), -127.0, 127.0).astype(jnp.int8)
        return q, scale

    def body(x_ref, w_ref, sx_ref, sw_ref, out_ref,
             fwd_ref, s1s_ref, s1r_ref, s2s_ref, s2r_ref,
             scs_ref, scr_ref, send_sems, recv_sems):
        my_i = lax.axis_index("i")
        p1 = jnp.bitwise_xor(my_i, 1)
        p2 = (N_DEV - 1) - my_i

        sa = jnp.minimum(p1, (N_DEV - 1) - p1)
        sb = jnp.maximum(p1, (N_DEV - 1) - p1)
        slot_fwd = jnp.where(my_i < 2, 1, 0)
        slot_own = 1 - slot_fwd

        w_q = w_ref[:, :].astype(DOT_DTYPE)

        def pchunk(c):
            a = x_ref[pl.ds(c * mc, mc), :].astype(DOT_DTYPE)
            return jnp.dot(a, w_q, preferred_element_type=jnp.float32)

        qa, sc_a = quantize(pchunk(sa))
        s1s_ref[0, :, :] = qa
        scs_ref[0, :, :] = jnp.full((8, 128), sc_a, jnp.float32)
        qb, sc_b = quantize(pchunk(sb))
        s1s_ref[1, :, :] = qb
        scs_ref[1, :, :] = jnp.full((8, 128), sc_b, jnp.float32)

        barrier = pltpu.get_barrier_semaphore()
        pl.semaphore_signal(barrier, inc=1, device_id=(p1,),
                            device_id_type=pl.DeviceIdType.MESH)
        pl.semaphore_signal(barrier, inc=1, device_id=(p2,),
                            device_id_type=pl.DeviceIdType.MESH)
        pl.semaphore_wait(barrier, 2)

        rdma1 = pltpu.make_async_remote_copy(
            src_ref=s1s_ref, dst_ref=s1r_ref,
            send_sem=send_sems.at[0], recv_sem=recv_sems.at[0],
            device_id=(p1,), device_id_type=pl.DeviceIdType.MESH)
        rdma1.start()
        rdma1_sc = pltpu.make_async_remote_copy(
            src_ref=scs_ref.at[pl.ds(0, 2)], dst_ref=scr_ref.at[pl.ds(0, 2)],
            send_sem=send_sems.at[1], recv_sem=recv_sems.at[1],
            device_id=(p1,), device_id_type=pl.DeviceIdType.MESH)
        rdma1_sc.start()

        fwd_ref[:, :] = pchunk((N_DEV - 1) - my_i)

        rdma1.wait()
        rdma1_sc.wait()
        fwd_ref[:, :] = fwd_ref[:, :] + (
            s1r_ref[pl.ds(slot_fwd, 1), :, :][0].astype(jnp.float32)
            * scr_ref[pl.ds(slot_fwd, 1), 0, 0][0])

        q2, sc_2 = quantize(fwd_ref[:, :])
        s2s_ref[:, :] = q2
        scs_ref[2, :, :] = jnp.full((8, 128), sc_2, jnp.float32)

        rdma2 = pltpu.make_async_remote_copy(
            src_ref=s2s_ref, dst_ref=s2r_ref,
            send_sem=send_sems.at[2], recv_sem=recv_sems.at[2],
            device_id=(p2,), device_id_type=pl.DeviceIdType.MESH)
        rdma2.start()
        rdma2_sc = pltpu.make_async_remote_copy(
            src_ref=scs_ref.at[pl.ds(2, 1)], dst_ref=scr_ref.at[pl.ds(2, 1)],
            send_sem=send_sems.at[3], recv_sem=recv_sems.at[3],
            device_id=(p2,), device_id_type=pl.DeviceIdType.MESH)
        rdma2_sc.start()

        out_ref[:, :] = pchunk(my_i) + (
            s1r_ref[pl.ds(slot_own, 1), :, :][0].astype(jnp.float32)
            * scr_ref[pl.ds(slot_own, 1), 0, 0][0])

        rdma2.wait()
        rdma2_sc.wait()
        out_ref[:, :] = (
            out_ref[:, :]
            + s2r_ref[:, :].astype(jnp.float32) * scr_ref[2, 0, 0]
        ) * (sx_ref[0] * sw_ref[0])

    return pl.pallas_call(
        body,
        out_shape=jax.ShapeDtypeStruct((mc, n), jnp.float32),
        in_specs=[
            pl.BlockSpec(memory_space=pltpu.VMEM),
            pl.BlockSpec(memory_space=pltpu.VMEM),
            pl.BlockSpec(memory_space=pltpu.SMEM),
            pl.BlockSpec(memory_space=pltpu.SMEM),
        ],
        out_specs=pl.BlockSpec(memory_space=pltpu.VMEM),
        scratch_shapes=[
            pltpu.VMEM((mc, n), jnp.float32),
            pltpu.VMEM((2, mc, n), qdtype),
            pltpu.VMEM((2, mc, n), qdtype),
            pltpu.VMEM((mc, n), qdtype),
            pltpu.VMEM((mc, n), qdtype),
            pltpu.VMEM((3, 8, 128), jnp.float32),
            pltpu.VMEM((3, 8, 128), jnp.float32),
            pltpu.SemaphoreType.DMA((4,)),
            pltpu.SemaphoreType.DMA((4,)),
        ],
        compiler_params=pltpu.CompilerParams(
            collective_id=0, vmem_limit_bytes=100 * 1024 * 1024),
    )(x, w_mat, scale_x, scale_w)


# device time: 80815 ns/iter; 1.3351x vs baseline; 1.3351x over previous
import jax
import jax.numpy as jnp
from jax import lax
from jax.experimental import pallas as pl
from jax.experimental.pallas import tpu as pltpu

N_DEV = 4

COMM_INT8 = True
DOT_DTYPE = jnp.float8_e5m2

_A, _B, _C, _A_SC, _B_SC, _C_SC = range(6)


def kernel(x, w_mat, scale_x, scale_w):
    m, _ = x.shape
    _, n = w_mat.shape
    mc = m // N_DEV

    qdtype = jnp.int8 if COMM_INT8 else jnp.float32

    def quantize(v):
        if not COMM_INT8:
            return v, jnp.float32(1.0)
        scale = jnp.maximum(jnp.max(jnp.abs(v)), 1e-30) / 127.0
        q = jnp.clip(jnp.round(v / scale), -127.0, 127.0).astype(jnp.int8)
        return q, scale

    def body(x_ref, w_ref, sx_ref, sw_ref, out_ref,
             fwd_ref, s1s_ref, s1r_ref, s2s_ref, s2r_ref,
             scs_ref, scr_ref, send_sems, recv_sems):
        my_i = lax.axis_index("i")
        p1 = jnp.bitwise_xor(my_i, 1)
        p2 = (N_DEV - 1) - my_i

        w_q = w_ref[:, :].astype(DOT_DTYPE)

        def pchunk(c):
            a = x_ref[pl.ds(c * mc, mc), :].astype(DOT_DTYPE)
            return jnp.dot(a, w_q, preferred_element_type=jnp.float32)

        def start_pair(data_rdma_args, slot, target):
            src, dst = data_rdma_args
            data = pltpu.make_async_remote_copy(
                src_ref=src, dst_ref=dst,
                send_sem=send_sems.at[slot], recv_sem=recv_sems.at[slot],
                device_id=(target,), device_id_type=pl.DeviceIdType.MESH)
            data.start()
            sc = pltpu.make_async_remote_copy(
                src_ref=scs_ref.at[pl.ds(slot, 1)],
                dst_ref=scr_ref.at[pl.ds(slot, 1)],
                send_sem=send_sems.at[slot + 3],
                recv_sem=recv_sems.at[slot + 3],
                device_id=(target,), device_id_type=pl.DeviceIdType.MESH)
            sc.start()
            return data, sc

        qA, scA = quantize(pchunk((N_DEV - 1) - p1))
        s1s_ref[0, :, :] = qA
        scs_ref[_A, :, :] = jnp.full((8, 128), scA, jnp.float32)

        barrier = pltpu.get_barrier_semaphore()
        pl.semaphore_signal(barrier, inc=1, device_id=(p1,),
                            device_id_type=pl.DeviceIdType.MESH)
        pl.semaphore_signal(barrier, inc=1, device_id=(p2,),
                            device_id_type=pl.DeviceIdType.MESH)
        pl.semaphore_wait(barrier, 2)

        rdmaA, rdmaA_sc = start_pair(
            (s1s_ref.at[0], s1r_ref.at[0]), _A, p1)

        qB, scB = quantize(pchunk(p1))
        s1s_ref[1, :, :] = qB
        scs_ref[_B, :, :] = jnp.full((8, 128), scB, jnp.float32)
        rdmaB, rdmaB_sc = start_pair(
            (s1s_ref.at[1], s1r_ref.at[1]), _B, p1)

        fwd_ref[:, :] = pchunk((N_DEV - 1) - my_i)

        rdmaA.wait()
        rdmaA_sc.wait()
        fwd_ref[:, :] = fwd_ref[:, :] + (
            s1r_ref[0].astype(jnp.float32) * scr_ref[_A, 0, 0])
        qC, scC = quantize(fwd_ref[:, :])
        s2s_ref[:, :] = qC
        scs_ref[_C, :, :] = jnp.full((8, 128), scC, jnp.float32)
        rdmaC, rdmaC_sc = start_pair((s2s_ref, s2r_ref), _C, p2)

        out_ref[:, :] = pchunk(my_i)

        rdmaB.wait()
        rdmaB_sc.wait()
        out_ref[:, :] = out_ref[:, :] + (
            s1r_ref[1].astype(jnp.float32) * scr_ref[_B, 0, 0])

        rdmaC.wait()
        rdmaC_sc.wait()
        out_ref[:, :] = (
            out_ref[:, :]
            + s2r_ref[:, :].astype(jnp.float32) * scr_ref[_C, 0, 0]
        ) * (sx_ref[0] * sw_ref[0])

    return pl.pallas_call(
        body,
        out_shape=jax.ShapeDtypeStruct((mc, n), jnp.float32),
        in_specs=[
            pl.BlockSpec(memory_space=pltpu.VMEM),
            pl.BlockSpec(memory_space=pltpu.VMEM),
            pl.BlockSpec(memory_space=pltpu.SMEM),
            pl.BlockSpec(memory_space=pltpu.SMEM),
        ],
        out_specs=pl.BlockSpec(memory_space=pltpu.VMEM),
        scratch_shapes=[
            pltpu.VMEM((mc, n), jnp.float32),
            pltpu.VMEM((2, mc, n), qdtype),
            pltpu.VMEM((2, mc, n), qdtype),
            pltpu.VMEM((mc, n), qdtype),
            pltpu.VMEM((mc, n), qdtype),
            pltpu.VMEM((6, 8, 128), jnp.float32),
            pltpu.VMEM((6, 8, 128), jnp.float32),
            pltpu.SemaphoreType.DMA((6,)),
            pltpu.SemaphoreType.DMA((6,)),
        ],
        compiler_params=pltpu.CompilerParams(
            collective_id=0, vmem_limit_bytes=100 * 1024 * 1024),
    )(x, w_mat, scale_x, scale_w)


# device time: 75852 ns/iter; 1.4225x vs baseline; 1.0654x over previous
import jax
import jax.numpy as jnp
from jax import lax
from jax.experimental import pallas as pl
from jax.experimental.pallas import tpu as pltpu

N_DEV = 4

COMM_INT8 = True
DOT_DTYPE = jnp.float8_e5m2

_A0, _A1, _B, _C0, _C1 = range(5)
_NSLOT = 5


def kernel(x, w_mat, scale_x, scale_w):
    m, _ = x.shape
    _, n = w_mat.shape
    mc = m // N_DEV
    hm = mc // 2

    qdtype = jnp.int8 if COMM_INT8 else jnp.float32

    def quantize(v):
        if not COMM_INT8:
            return v, jnp.float32(1.0)
        scale = jnp.maximum(jnp.max(jnp.abs(v)), 1e-30) / 127.0
        q = jnp.clip(jnp.round(v / scale), -127.0, 127.0).astype(jnp.int8)
        return q, scale

    def body(x_ref, w_ref, sx_ref, sw_ref, out_ref,
             fwd_ref, sAs, sAr, sBs, sBr, sCs, sCr,
             scs_ref, scr_ref, send_sems, recv_sems):
        my_i = lax.axis_index("i")
        p1 = jnp.bitwise_xor(my_i, 1)
        p2 = (N_DEV - 1) - my_i

        w_q = w_ref[:, :].astype(DOT_DTYPE)

        def phalf(c, h):
            a = x_ref[pl.ds(c * mc + h * hm, hm), :].astype(DOT_DTYPE)
            return jnp.dot(a, w_q, preferred_element_type=jnp.float32)

        def qstore(dst, h, v, slot):
            q, sc = quantize(v)
            dst[pl.ds(h * hm, hm), :] = q
            scs_ref[slot, :, :] = jnp.full((8, 128), sc, jnp.float32)

        def send_pair(src, dst, slot, h, nrows, target):
            data = pltpu.make_async_remote_copy(
                src_ref=src.at[pl.ds(h * hm, nrows)],
                dst_ref=dst.at[pl.ds(h * hm, nrows)],
                send_sem=send_sems.at[slot], recv_sem=recv_sems.at[slot],
                device_id=(target,), device_id_type=pl.DeviceIdType.MESH)
            data.start()
            sc = pltpu.make_async_remote_copy(
                src_ref=scs_ref.at[pl.ds(slot, 1)],
                dst_ref=scr_ref.at[pl.ds(slot, 1)],
                send_sem=send_sems.at[slot + _NSLOT],
                recv_sem=recv_sems.at[slot + _NSLOT],
                device_id=(target,), device_id_type=pl.DeviceIdType.MESH)
            sc.start()
            return data, sc

        def deq(recv, h, nrows, slot):
            return (recv[pl.ds(h * hm, nrows), :].astype(jnp.float32)
                    * scr_ref[slot, 0, 0])

        cA = (N_DEV - 1) - p1
        cF = (N_DEV - 1) - my_i

        qstore(sAs, 0, phalf(cA, 0), _A0)

        barrier = pltpu.get_barrier_semaphore()
        pl.semaphore_signal(barrier, inc=1, device_id=(p1,),
                            device_id_type=pl.DeviceIdType.MESH)
        pl.semaphore_signal(barrier, inc=1, device_id=(p2,),
                            device_id_type=pl.DeviceIdType.MESH)
        pl.semaphore_wait(barrier, 2)

        rA0 = send_pair(sAs, sAr, _A0, 0, hm, p1)

        qstore(sAs, 1, phalf(cA, 1), _A1)
        rA1 = send_pair(sAs, sAr, _A1, 1, hm, p1)

        aB = x_ref[pl.ds(p1 * mc, mc), :].astype(DOT_DTYPE)
        qB, scB = quantize(
            jnp.dot(aB, w_q, preferred_element_type=jnp.float32))
        sBs[:, :] = qB
        scs_ref[_B, :, :] = jnp.full((8, 128), scB, jnp.float32)
        rB = send_pair(sBs, sBr, _B, 0, mc, p1)

        fwd_ref[pl.ds(0, hm), :] = phalf(cF, 0)
        for r in rA0:
            r.wait()
        qstore(sCs, 0, fwd_ref[pl.ds(0, hm), :] + deq(sAr, 0, hm, _A0), _C0)
        rC0 = send_pair(sCs, sCr, _C0, 0, hm, p2)

        fwd_ref[pl.ds(hm, hm), :] = phalf(cF, 1)
        for r in rA1:
            r.wait()
        qstore(sCs, 1, fwd_ref[pl.ds(hm, hm), :] + deq(sAr, 1, hm, _A1), _C1)
        rC1 = send_pair(sCs, sCr, _C1, 1, hm, p2)

        out_ref[pl.ds(0, hm), :] = phalf(my_i, 0)
        out_ref[pl.ds(hm, hm), :] = phalf(my_i, 1)

        for r in rB:
            r.wait()
        out_ref[:, :] = out_ref[:, :] + deq(sBr, 0, mc, _B)

        s = sx_ref[0] * sw_ref[0]
        for r in rC0:
            r.wait()
        out_ref[pl.ds(0, hm), :] = (
            out_ref[pl.ds(0, hm), :] + deq(sCr, 0, hm, _C0)) * s
        for r in rC1:
            r.wait()
        out_ref[pl.ds(hm, hm), :] = (
            out_ref[pl.ds(hm, hm), :] + deq(sCr, 1, hm, _C1)) * s

    return pl.pallas_call(
        body,
        out_shape=jax.ShapeDtypeStruct((mc, n), jnp.float32),
        in_specs=[
            pl.BlockSpec(memory_space=pltpu.VMEM),
            pl.BlockSpec(memory_space=pltpu.VMEM),
            pl.BlockSpec(memory_space=pltpu.SMEM),
            pl.BlockSpec(memory_space=pltpu.SMEM),
        ],
        out_specs=pl.BlockSpec(memory_space=pltpu.VMEM),
        scratch_shapes=[
            pltpu.VMEM((mc, n), jnp.float32),
            pltpu.VMEM((mc, n), qdtype),
            pltpu.VMEM((mc, n), qdtype),
            pltpu.VMEM((mc, n), qdtype),
            pltpu.VMEM((mc, n), qdtype),
            pltpu.VMEM((mc, n), qdtype),
            pltpu.VMEM((mc, n), qdtype),
            pltpu.VMEM((_NSLOT, 8, 128), jnp.float32),
            pltpu.VMEM((_NSLOT, 8, 128), jnp.float32),
            pltpu.SemaphoreType.DMA((2 * _NSLOT,)),
            pltpu.SemaphoreType.DMA((2 * _NSLOT,)),
        ],
        compiler_params=pltpu.CompilerParams(
            collective_id=0, vmem_limit_bytes=100 * 1024 * 1024),
    )(x, w_mat, scale_x, scale_w)


# device time: 74838 ns/iter; 1.4418x vs baseline; 1.0135x over previous
import jax
import jax.numpy as jnp
from jax import lax
from jax.experimental import pallas as pl
from jax.experimental.pallas import tpu as pltpu

N_DEV = 4

COMM_INT8 = True
DOT_DTYPE = jnp.float8_e5m2

PIECES = ((0, 256), (256, 512), (768, 256))
_NP = len(PIECES)
_SLOT_A = 0
_SLOT_B = _NP
_SLOT_C = _NP + 1
_NSLOT = 2 * _NP + 1


def kernel(x, w_mat, scale_x, scale_w):
    m, _ = x.shape
    _, n = w_mat.shape
    mc = m // N_DEV

    qdtype = jnp.int8 if COMM_INT8 else jnp.float32

    def quantize(v):
        if not COMM_INT8:
            return v, jnp.float32(1.0)
        scale = jnp.maximum(jnp.max(jnp.abs(v)), 1e-30) / 127.0
        q = jnp.clip(jnp.round(v / scale), -127.0, 127.0).astype(jnp.int8)
        return q, scale

    def body(x_ref, w_ref, sx_ref, sw_ref, out_ref,
             fwd_ref, sAs, sAr, sBs, sBr, sCs, sCr,
             scs_ref, scr_ref, send_sems, recv_sems):
        my_i = lax.axis_index("i")
        p1 = jnp.bitwise_xor(my_i, 1)
        p2 = (N_DEV - 1) - my_i

        w_q = w_ref[:, :].astype(DOT_DTYPE)

        def pdot(row0, nrows):
            a = x_ref[pl.ds(row0, nrows), :].astype(DOT_DTYPE)
            return jnp.dot(a, w_q, preferred_element_type=jnp.float32)

        def qstore(dst, st, sz, v, slot):
            q, sc = quantize(v)
            dst[pl.ds(st, sz), :] = q
            scs_ref[slot, :, :] = jnp.full((8, 128), sc, jnp.float32)

        def send_pair(src, dst, slot, st, sz, target):
            data = pltpu.make_async_remote_copy(
                src_ref=src.at[pl.ds(st, sz)],
                dst_ref=dst.at[pl.ds(st, sz)],
                send_sem=send_sems.at[slot], recv_sem=recv_sems.at[slot],
                device_id=(target,), device_id_type=pl.DeviceIdType.MESH)
            data.start()
            sc = pltpu.make_async_remote_copy(
                src_ref=scs_ref.at[pl.ds(slot, 1)],
                dst_ref=scr_ref.at[pl.ds(slot, 1)],
                send_sem=send_sems.at[slot + _NSLOT],
                recv_sem=recv_sems.at[slot + _NSLOT],
                device_id=(target,), device_id_type=pl.DeviceIdType.MESH)
            sc.start()
            return data, sc

        def deq(recv, st, sz, slot):
            return (recv[pl.ds(st, sz), :].astype(jnp.float32)
                    * scr_ref[slot, 0, 0])

        cA = (N_DEV - 1) - p1
        cF = (N_DEV - 1) - my_i

        rA = []
        for k, (st, sz) in enumerate(PIECES):
            qstore(sAs, st, sz, pdot(cA * mc + st, sz), _SLOT_A + k)
            if k == 0:
                barrier = pltpu.get_barrier_semaphore()
                pl.semaphore_signal(barrier, inc=1, device_id=(p1,),
                                    device_id_type=pl.DeviceIdType.MESH)
                pl.semaphore_signal(barrier, inc=1, device_id=(p2,),
                                    device_id_type=pl.DeviceIdType.MESH)
                pl.semaphore_wait(barrier, 2)
            rA.append(send_pair(sAs, sAr, _SLOT_A + k, st, sz, p1))

        qstore(sBs, 0, mc, pdot(p1 * mc, mc), _SLOT_B)
        rB = send_pair(sBs, sBr, _SLOT_B, 0, mc, p1)

        rC = []
        for k, (st, sz) in enumerate(PIECES):
            fwd_ref[pl.ds(st, sz), :] = pdot(cF * mc + st, sz)
            for r in rA[k]:
                r.wait()
            qstore(sCs, st, sz,
                   fwd_ref[pl.ds(st, sz), :] + deq(sAr, st, sz, _SLOT_A + k),
                   _SLOT_C + k)
            rC.append(send_pair(sCs, sCr, _SLOT_C + k, st, sz, p2))

        out_ref[:, :] = pdot(my_i * mc, mc)

        for r in rB:
            r.wait()
        out_ref[:, :] = out_ref[:, :] + deq(sBr, 0, mc, _SLOT_B)

        s = sx_ref[0] * sw_ref[0]
        for k, (st, sz) in enumerate(PIECES):
            for r in rC[k]:
                r.wait()
            out_ref[pl.ds(st, sz), :] = (
                out_ref[pl.ds(st, sz), :] + deq(sCr, st, sz, _SLOT_C + k)) * s

    return pl.pallas_call(
        body,
        out_shape=jax.ShapeDtypeStruct((mc, n), jnp.float32),
        in_specs=[
            pl.BlockSpec(memory_space=pltpu.VMEM),
            pl.BlockSpec(memory_space=pltpu.VMEM),
            pl.BlockSpec(memory_space=pltpu.SMEM),
            pl.BlockSpec(memory_space=pltpu.SMEM),
        ],
        out_specs=pl.BlockSpec(memory_space=pltpu.VMEM),
        scratch_shapes=[
            pltpu.VMEM((mc, n), jnp.float32),
            pltpu.VMEM((mc, n), qdtype),
            pltpu.VMEM((mc, n), qdtype),
            pltpu.VMEM((mc, n), qdtype),
            pltpu.VMEM((mc, n), qdtype),
            pltpu.VMEM((mc, n), qdtype),
            pltpu.VMEM((mc, n), qdtype),
            pltpu.VMEM((_NSLOT, 8, 128), jnp.float32),
            pltpu.VMEM((_NSLOT, 8, 128), jnp.float32),
            pltpu.SemaphoreType.DMA((2 * _NSLOT,)),
            pltpu.SemaphoreType.DMA((2 * _NSLOT,)),
        ],
        compiler_params=pltpu.CompilerParams(
            collective_id=0, vmem_limit_bytes=100 * 1024 * 1024),
    )(x, w_mat, scale_x, scale_w)


# device time: 73528 ns/iter; 1.4675x vs baseline; 1.0178x over previous
import contextlib

import jax
import jax.numpy as jnp
from jax import lax
from jax.experimental import pallas as pl
from jax.experimental.pallas import tpu as pltpu

N_DEV = 4

PROFILE_SCOPES = False


def _scope(name):
    if PROFILE_SCOPES:
        return jax.named_scope(name)
    return contextlib.nullcontext()

COMM_INT8 = True
DOT_DTYPE = jnp.float8_e5m2

PIECES = ((0, 256), (256, 512), (768, 256))
_NP = len(PIECES)
_SLOT_A = 0
_SLOT_B = _NP
_SLOT_C = _NP + 1
_NSLOT = 2 * _NP + 1


def kernel(x, w_mat, scale_x, scale_w):
    m, _ = x.shape
    _, n = w_mat.shape
    mc = m // N_DEV

    qdtype = jnp.int8 if COMM_INT8 else jnp.float32

    def quantize(v):
        if not COMM_INT8:
            return v, jnp.float32(1.0)
        scale = jnp.maximum(jnp.max(jnp.abs(v)), 1e-30) / 127.0
        q = jnp.clip(jnp.round(v / scale), -127.0, 127.0).astype(jnp.int8)
        return q, scale

    def body(x_ref, w_ref, sx_ref, sw_ref, out_ref,
             acc_ref, fwd_ref, sAs, sAr, sBs, sBr, sCs, sCr,
             scs_ref, scr_ref, send_sems, recv_sems, out_sems):
        my_i = lax.axis_index("i")
        p1 = jnp.bitwise_xor(my_i, 1)
        p2 = (N_DEV - 1) - my_i

        w_q = w_ref[:, :].astype(DOT_DTYPE)

        def pdot(row0, nrows):
            a = x_ref[pl.ds(row0, nrows), :].astype(DOT_DTYPE)
            return jnp.dot(a, w_q, preferred_element_type=jnp.float32)

        def qstore(dst, st, sz, v, slot):
            q, sc = quantize(v)
            dst[pl.ds(st, sz), :] = q
            scs_ref[slot, :, :] = jnp.full((8, 128), sc, jnp.float32)

        def send_pair(src, dst, slot, st, sz, target):
            data = pltpu.make_async_remote_copy(
                src_ref=src.at[pl.ds(st, sz)],
                dst_ref=dst.at[pl.ds(st, sz)],
                send_sem=send_sems.at[slot], recv_sem=recv_sems.at[slot],
                device_id=(target,), device_id_type=pl.DeviceIdType.MESH)
            data.start()
            sc = pltpu.make_async_remote_copy(
                src_ref=scs_ref.at[pl.ds(slot, 1)],
                dst_ref=scr_ref.at[pl.ds(slot, 1)],
                send_sem=send_sems.at[slot + _NSLOT],
                recv_sem=recv_sems.at[slot + _NSLOT],
                device_id=(target,), device_id_type=pl.DeviceIdType.MESH)
            sc.start()
            return data, sc

        def deq(recv, st, sz, slot):
            return (recv[pl.ds(st, sz), :].astype(jnp.float32)
                    * scr_ref[slot, 0, 0])

        cA = (N_DEV - 1) - p1
        cF = (N_DEV - 1) - my_i

        rA = []
        for k, (st, sz) in enumerate(PIECES):
            with _scope(f"computeA#k={k}"):
                qstore(sAs, st, sz, pdot(cA * mc + st, sz), _SLOT_A + k)
            if k == 0:
                with _scope("barrier"):
                    barrier = pltpu.get_barrier_semaphore()
                    pl.semaphore_signal(barrier, inc=1, device_id=(p1,),
                                        device_id_type=pl.DeviceIdType.MESH)
                    pl.semaphore_signal(barrier, inc=1, device_id=(p2,),
                                        device_id_type=pl.DeviceIdType.MESH)
                    pl.semaphore_wait(barrier, 2)
            rA.append(send_pair(sAs, sAr, _SLOT_A + k, st, sz, p1))

        with _scope("computeB"):
            qstore(sBs, 0, mc, pdot(p1 * mc, mc), _SLOT_B)
        rB = send_pair(sBs, sBr, _SLOT_B, 0, mc, p1)

        rC = []
        for k, (st, sz) in enumerate(PIECES):
            with _scope(f"computeF#k={k}"):
                fwd_ref[pl.ds(st, sz), :] = pdot(cF * mc + st, sz)
            with _scope(f"waitA#k={k}"):
                for r in rA[k]:
                    r.wait()
            with _scope(f"buildC#k={k}"):
                qstore(sCs, st, sz,
                       fwd_ref[pl.ds(st, sz), :]
                       + deq(sAr, st, sz, _SLOT_A + k),
                       _SLOT_C + k)
            rC.append(send_pair(sCs, sCr, _SLOT_C + k, st, sz, p2))

        with _scope("computeOwn"):
            acc_ref[:, :] = pdot(my_i * mc, mc)

        with _scope("waitB"):
            for r in rB:
                r.wait()
        with _scope("addB"):
            acc_ref[:, :] = acc_ref[:, :] + deq(sBr, 0, mc, _SLOT_B)

        s = sx_ref[0] * sw_ref[0]
        out_cps = []
        for k, (st, sz) in enumerate(PIECES):
            with _scope(f"waitC#k={k}"):
                for r in rC[k]:
                    r.wait()
            with _scope(f"epilogue#k={k}"):
                acc_ref[pl.ds(st, sz), :] = (
                    acc_ref[pl.ds(st, sz), :]
                    + deq(sCr, st, sz, _SLOT_C + k)) * s
                cp = pltpu.make_async_copy(
                    acc_ref.at[pl.ds(st, sz)],
                    out_ref.at[pl.ds(st, sz)],
                    out_sems.at[k])
                cp.start()
                out_cps.append(cp)
        with _scope("writeout_wait"):
            for cp in out_cps:
                cp.wait()

    return pl.pallas_call(
        body,
        out_shape=jax.ShapeDtypeStruct((mc, n), jnp.float32),
        in_specs=[
            pl.BlockSpec(memory_space=pltpu.VMEM),
            pl.BlockSpec(memory_space=pltpu.VMEM),
            pl.BlockSpec(memory_space=pltpu.SMEM),
            pl.BlockSpec(memory_space=pltpu.SMEM),
        ],
        out_specs=pl.BlockSpec(memory_space=pl.ANY),
        scratch_shapes=[
            pltpu.VMEM((mc, n), jnp.float32),
            pltpu.VMEM((mc, n), jnp.float32),
            pltpu.VMEM((mc, n), qdtype),
            pltpu.VMEM((mc, n), qdtype),
            pltpu.VMEM((mc, n), qdtype),
            pltpu.VMEM((mc, n), qdtype),
            pltpu.VMEM((mc, n), qdtype),
            pltpu.VMEM((mc, n), qdtype),
            pltpu.VMEM((_NSLOT, 8, 128), jnp.float32),
            pltpu.VMEM((_NSLOT, 8, 128), jnp.float32),
            pltpu.SemaphoreType.DMA((2 * _NSLOT,)),
            pltpu.SemaphoreType.DMA((2 * _NSLOT,)),
            pltpu.SemaphoreType.DMA((_NP,)),
        ],
        compiler_params=pltpu.CompilerParams(
            collective_id=0, vmem_limit_bytes=100 * 1024 * 1024),
    )(x, w_mat, scale_x, scale_w)
